# baseline (speedup 1.0000x reference)
import functools

import jax
import jax.numpy as jnp
from jax import lax
from jax.experimental import pallas as pl
from jax.experimental.pallas import tpu as pltpu
from jax.experimental.pallas import tpu_sc as plsc

_L = 16
_NW = 32
_S = 10
_C = 64


@functools.lru_cache(maxsize=None)
def _make_sc_agg(Bp: int, D: int):
    bpw = Bp // _NW
    n_chunks = bpw // _C
    mesh = plsc.VectorSubcoreMesh(core_axis_name="c", subcore_axis_name="s")

    @functools.partial(
        pl.kernel,
        mesh=mesh,
        out_type=jax.ShapeDtypeStruct((Bp, D), jnp.float32),
        scratch_types=[
            pltpu.VMEM((bpw,), jnp.int32),
            pltpu.VMEM((_C, D), jnp.float32),
            pltpu.SemaphoreType.DMA,
        ],
    )
    def k(features_hbm, idx_hbm, out_hbm, idx_v, acc_v, sem):
        cid = lax.axis_index("c")
        sid = lax.axis_index("s")
        wid = sid * 2 + cid
        w_node_base = wid * bpw

        def chunk_body(ci, carry):
            nbase = w_node_base + ci * _C

            def zero_body(n, c2):
                for d in range(D // _L):
                    acc_v[n, pl.ds(d * _L, _L)] = jnp.zeros((_L,), jnp.float32)
                return c2

            lax.fori_loop(0, _C, zero_body, 0)
            for s in range(_S):
                pltpu.sync_copy(
                    idx_hbm.at[pl.ds(s * Bp + nbase, _C)], idx_v.at[pl.ds(0, _C)]
                )
                pltpu.sync_copy(
                    features_hbm.at[idx_v.at[pl.ds(0, _C)]], acc_v, add=True
                )

            def scale_body(n, c2):
                for d in range(D // _L):
                    acc_v[n, pl.ds(d * _L, _L)] = acc_v[n, pl.ds(d * _L, _L)] * (
                        1.0 / _S
                    )
                return c2

            lax.fori_loop(0, _C, scale_body, 0)
            pltpu.sync_copy(acc_v, out_hbm.at[pl.ds(nbase, _C)])
            return carry

        lax.fori_loop(0, n_chunks, chunk_body, 0)

    return k


def kernel(nodes, neigh_idx, features):
    B, S = neigh_idx.shape
    D = features.shape[1]
    block = _NW * _C
    Bp = ((B + block - 1) // block) * block
    idx = neigh_idx.astype(jnp.int32)
    if Bp != B:
        idx = jnp.pad(idx, ((0, Bp - B), (0, 0)))
    idxT = idx.T.reshape(_S * Bp)
    out = _make_sc_agg(Bp, D)(features, idxT)
    return out[:B]


# speedup vs baseline: 2.0543x; 2.0543x over previous
import functools

import jax
import jax.numpy as jnp
from jax import lax
from jax.experimental import pallas as pl
from jax.experimental.pallas import tpu as pltpu
from jax.experimental.pallas import tpu_sc as plsc

_L = 16
_NW = 32
_S = 10
_C = 112


@functools.lru_cache(maxsize=None)
def _make_sc_agg(Bp: int, D: int):
    bpw = Bp // _NW
    n_pairs = bpw // (2 * _C)
    mesh = plsc.VectorSubcoreMesh(core_axis_name="c", subcore_axis_name="s")

    @functools.partial(
        pl.kernel,
        mesh=mesh,
        out_type=jax.ShapeDtypeStruct((Bp, D), jnp.float32),
        scratch_types=[
            pltpu.VMEM((_S * bpw,), jnp.int32),
            pltpu.VMEM((2, _C, D), jnp.float32),
            pltpu.SemaphoreType.DMA,
            pltpu.SemaphoreType.DMA,
        ],
    )
    def k(features_hbm, idx_hbm, out_hbm, idx_v, acc_v, sem0, sem1):
        cid = lax.axis_index("c")
        sid = lax.axis_index("s")
        wid = sid * 2 + cid
        w_node_base = wid * bpw
        sems = (sem0, sem1)

        pltpu.sync_copy(idx_hbm.at[pl.ds(wid * (_S * bpw), _S * bpw)], idx_v)

        def zero(buf):
            def zbody(n, c2):
                for d in range(D // _L):
                    acc_v[buf, n, pl.ds(d * _L, _L)] = jnp.zeros((_L,), jnp.float32)
                return c2

            lax.fori_loop(0, _C, zbody, 0)

        def fire(ci, buf):
            for s in range(_S):
                pltpu.async_copy(
                    features_hbm.at[idx_v.at[pl.ds(s * bpw + ci * _C, _C)]],
                    acc_v.at[buf],
                    sems[buf],
                    add=True,
                )

        def drain(buf):
            for _ in range(_S):
                pltpu.make_async_copy(
                    features_hbm.at[pl.ds(0, _C)], acc_v.at[buf], sems[buf]
                ).wait()

        def scale_store(ci, buf):
            def sbody(n, c2):
                for d in range(D // _L):
                    acc_v[buf, n, pl.ds(d * _L, _L)] = acc_v[
                        buf, n, pl.ds(d * _L, _L)
                    ] * (1.0 / _S)
                return c2

            lax.fori_loop(0, _C, sbody, 0)
            nbase = w_node_base + ci * _C
            pltpu.sync_copy(acc_v.at[buf], out_hbm.at[pl.ds(nbase, _C)])

        zero(0)
        fire(0, 0)

        def pair_body(g, carry):
            c0 = 2 * g
            zero(1)
            fire(c0 + 1, 1)
            drain(0)
            scale_store(c0, 0)

            @pl.when(g < n_pairs - 1)
            def _():
                zero(0)
                fire(c0 + 2, 0)

            drain(1)
            scale_store(c0 + 1, 1)
            return carry

        lax.fori_loop(0, n_pairs, pair_body, 0)

    return k


def kernel(nodes, neigh_idx, features):
    B, S = neigh_idx.shape
    D = features.shape[1]
    block = 2 * _NW * _C
    Bp = ((B + block - 1) // block) * block
    idx = neigh_idx.astype(jnp.int32)
    if Bp != B:
        idx = jnp.pad(idx, ((0, Bp - B), (0, 0)))
    bpw = Bp // _NW
    idxT = idx.reshape(_NW, bpw, S).transpose(0, 2, 1).reshape(_NW * S * bpw)
    out = _make_sc_agg(Bp, D)(features, idxT)
    return out[:B]
